# baseline (speedup 1.0000x reference)
import functools

import numpy as np
import jax
import jax.numpy as jnp
from jax import lax
from jax.experimental import pallas as pl
from jax.experimental.pallas import tpu as pltpu
from jax.experimental.pallas import tpu_sc as plsc

N = 10000
E = 320000
D = 128

NC = 2
NS = 16
NW = NC * NS
EP = E // NW
K = 80
SEG = 25
NSEG = EP // (SEG * K)
NPAD = 10240
NSLICE = NPAD // NS

_PERM = np.arange(D).reshape(4, 2, 16).transpose(0, 2, 1).reshape(-1)


def _rsqrt16(d):
    dm = jnp.maximum(d, 1.0)
    bits = lax.bitcast_convert_type(dm, jnp.int32)
    y = lax.bitcast_convert_type(jnp.int32(0x5F3759DF) - (bits >> 1),
                                 jnp.float32)
    for _ in range(3):
        y = y * (1.5 - 0.5 * dm * y * y)
    return jnp.where(d > 0.5, y, 0.0)


def _sc_body(xb_hbm, ei5, dist5, part_out,
             sh_out, sh_disr,
             disr_t, rowi, coli, vals_t, ringb, scaled,
             deg_t, ones_k, sem_g, sem_s, sem_d):
    _Z16 = jnp.zeros((16,), jnp.float32)
    _O16 = jnp.ones((16,), jnp.float32)
    c = lax.axis_index("c")
    s = lax.axis_index("s")
    w = 2 * s + c
    base = s * NSLICE

    @pl.loop(0, NSLICE // 16)
    def _(i):
        deg_t[pl.ds(i * 16, 16)] = _Z16

    @pl.loop(0, K)
    def _(r):
        for k in range(D // 16):
            scaled[r, pl.ds(k * 16, 16)] = _Z16

    for q in range(K // 16):
        ones_k[pl.ds(q * 16, 16)] = _O16

    pltpu.sync_copy(deg_t, sh_disr.at[pl.ds(base, NSLICE)])
    for j in range(NSLICE // K):
        pltpu.sync_copy(scaled, sh_out.at[pl.ds(base + j * K, K)])
    plsc.subcore_barrier()

    for j2 in range(2):
        @pl.loop(0, NSEG)
        def _(seg):
            pltpu.sync_copy(ei5.at[1, 2 * s + j2, seg], rowi)

            @pl.loop(0, SEG)
            def _(i):
                pltpu.async_copy(ones_k, sh_disr.at[rowi.at[i]], sem_d,
                                 add=True)

            @pl.loop(0, SEG)
            def _(i):
                pltpu.make_async_copy(ones_k, sh_disr.at[rowi.at[0]],
                                      sem_d).wait()

    plsc.subcore_barrier()

    pltpu.sync_copy(sh_disr.at[pl.ds(base, NSLICE)], deg_t)

    @pl.loop(0, NSLICE // 16)
    def _(i):
        sl = pl.ds(i * 16, 16)
        deg_t[sl] = _rsqrt16(deg_t[sl])

    pltpu.sync_copy(deg_t, sh_disr.at[pl.ds(base, NSLICE)])
    plsc.subcore_barrier()

    pltpu.sync_copy(sh_disr, disr_t)

    @pl.loop(0, NSEG)
    def _(seg):
        pltpu.sync_copy(ei5.at[0, w, seg], rowi)
        pltpu.sync_copy(ei5.at[1, w, seg], coli)
        pltpu.sync_copy(dist5.at[w, seg], vals_t)

        @pl.loop(0, SEG)
        def _(i):
            for q in range(K // 16):
                sl = pl.ds(q * 16, 16)
                dd = vals_t[i, sl]
                dr = plsc.load_gather(disr_t, [rowi[i, sl]])
                dc = plsc.load_gather(disr_t, [coli[i, sl]])
                vals_t[i, sl] = jnp.exp(-dd * dd) * dr * dc

        pltpu.async_copy(xb_hbm.at[coli.at[0]], ringb.at[0], sem_g)
        pltpu.async_copy(xb_hbm.at[coli.at[1]], ringb.at[1], sem_g)

        @pl.loop(0, SEG)
        def _(i):
            m = lax.rem(i, 3)

            @pl.when(i < SEG - 2)
            def _():
                pltpu.async_copy(xb_hbm.at[coli.at[i + 2]],
                                 ringb.at[lax.rem(i + 2, 3)], sem_g)

            pltpu.make_async_copy(xb_hbm.at[coli.at[i]], ringb.at[m],
                                  sem_g).wait()

            @pl.when(i >= 1)
            def _():
                pltpu.make_async_copy(scaled, sh_out.at[rowi.at[0]],
                                      sem_s).wait()

            for g in range(K // 16):
                vv = vals_t[i, pl.ds(g * 16, 16)]
                for j in range(16):
                    v = vv[j]
                    e = g * 16 + j
                    for k in range(D // 32):
                        wrd = plsc.bitcast(ringb[m, e, pl.ds(k * 32, 32)],
                                           jnp.int32)
                        lo = lax.bitcast_convert_type(wrd << 16, jnp.float32)
                        hi = lax.bitcast_convert_type(
                            wrd & jnp.int32(-65536), jnp.float32)
                        scaled[e, pl.ds(k * 32, 16)] = lo * v
                        scaled[e, pl.ds(k * 32 + 16, 16)] = hi * v

            pltpu.async_copy(scaled, sh_out.at[rowi.at[i]], sem_s,
                             add=True)

        pltpu.make_async_copy(scaled, sh_out.at[rowi.at[0]], sem_s).wait()

    plsc.subcore_barrier()

    pltpu.sync_copy(sh_out.at[pl.ds(base, NSLICE)],
                    part_out.at[c, pl.ds(base, NSLICE)])


_sc_call = functools.partial(
    pl.kernel,
    out_type=jax.ShapeDtypeStruct((NC, NPAD, D), jnp.float32),
    mesh=plsc.VectorSubcoreMesh(core_axis_name="c", subcore_axis_name="s",
                                num_cores=NC, num_subcores=NS),
    compiler_params=pltpu.CompilerParams(needs_layout_passes=False,
                                         use_tc_tiling_on_sc=False),
    scratch_types=[
        pltpu.VMEM_SHARED((NPAD, D), jnp.float32),
        pltpu.VMEM_SHARED((NPAD,), jnp.float32),
        pltpu.VMEM((NPAD,), jnp.float32),
        pltpu.VMEM((SEG, K), jnp.int32),
        pltpu.VMEM((SEG, K), jnp.int32),
        pltpu.VMEM((SEG, K), jnp.float32),
        pltpu.VMEM((3, K, D), jnp.bfloat16),
        pltpu.VMEM((K, D), jnp.float32),
        pltpu.VMEM((NSLICE,), jnp.float32),
        pltpu.VMEM((K,), jnp.float32),
        pltpu.SemaphoreType.DMA,
        pltpu.SemaphoreType.DMA,
        pltpu.SemaphoreType.DMA,
    ],
)(_sc_body)


def _mm_body(p_ref, wt_ref, b_ref, o_ref):
    sblk = p_ref[0] + p_ref[1]
    o_ref[...] = (
        jnp.dot(sblk, wt_ref[...], preferred_element_type=jnp.float32)
        + b_ref[...]
    )


def _mm_call(parts, wt, b2):
    return pl.pallas_call(
        _mm_body,
        grid=(NPAD // 512,),
        in_specs=[
            pl.BlockSpec((NC, 512, D), lambda i: (0, i, 0)),
            pl.BlockSpec((D, D), lambda i: (0, 0)),
            pl.BlockSpec((1, D), lambda i: (0, 0)),
        ],
        out_specs=pl.BlockSpec((512, D), lambda i: (i, 0)),
        out_shape=jax.ShapeDtypeStruct((NPAD, D), jnp.float32),
    )(parts, wt, b2)


@jax.jit
def kernel(x, edge_index, dist_vec, W, b):
    xb = jnp.take(x, _PERM, axis=1).astype(jnp.bfloat16)
    ei5 = edge_index.reshape(2, NW, NSEG, SEG, K)
    dist5 = dist_vec.reshape(NW, NSEG, SEG, K)
    parts = _sc_call(xb, ei5, dist5)
    out = _mm_call(parts, W.T, b.reshape(1, D))
    return out[:N]


# speedup vs baseline: 2.4354x; 2.4354x over previous
import functools

import jax
import jax.numpy as jnp
from jax import lax
from jax.experimental import pallas as pl
from jax.experimental.pallas import tpu as pltpu
from jax.experimental.pallas import tpu_sc as plsc

N = 10000
E = 320000
D = 128

NC = 2
NS = 16
NW = NC * NS
EP = E // NW
K = 80
SEG = 25
NSEG = EP // (SEG * K)
NPAD = 10240
NSLICE = NPAD // NS


def _rsqrt16(d):
    dm = jnp.maximum(d, 1.0)
    bits = lax.bitcast_convert_type(dm, jnp.int32)
    y = lax.bitcast_convert_type(jnp.int32(0x5F3759DF) - (bits >> 1),
                                 jnp.float32)
    for _ in range(3):
        y = y * (1.5 - 0.5 * dm * y * y)
    return jnp.where(d > 0.5, y, 0.0)


def _sc_body(x_hbm, ei5, dist5, part_out,
             sh_out, sh_disr,
             disr_t, rowi, coli, vals_t, ring,
             deg_t, ones_k, sem_g, sem_s, sem_d):
    _Z16 = jnp.zeros((16,), jnp.float32)
    _O16 = jnp.ones((16,), jnp.float32)
    c = lax.axis_index("c")
    s = lax.axis_index("s")
    w = 2 * s + c
    base = s * NSLICE

    @pl.loop(0, NSLICE // 16)
    def _(i):
        deg_t[pl.ds(i * 16, 16)] = _Z16

    @pl.loop(0, K)
    def _(r):
        for k in range(D // 16):
            ring[0, r, pl.ds(k * 16, 16)] = _Z16

    for q in range(K // 16):
        ones_k[pl.ds(q * 16, 16)] = _O16

    pltpu.sync_copy(deg_t, sh_disr.at[pl.ds(base, NSLICE)])
    for j in range(NSLICE // K):
        pltpu.sync_copy(ring.at[0], sh_out.at[pl.ds(base + j * K, K)])
    plsc.subcore_barrier()

    for seg in range(NSEG):
        pltpu.sync_copy(ei5.at[1, 2 * s, seg], rowi)
        pltpu.sync_copy(ei5.at[1, 2 * s + 1, seg], coli)

        @pl.loop(0, SEG)
        def _(i):
            pltpu.async_copy(ones_k, sh_disr.at[rowi.at[i]], sem_d, add=True)
            pltpu.async_copy(ones_k, sh_disr.at[coli.at[i]], sem_d, add=True)

        @pl.loop(0, 2 * SEG)
        def _(i):
            pltpu.make_async_copy(ones_k, sh_disr.at[coli.at[0]],
                                  sem_d).wait()

    plsc.subcore_barrier()

    pltpu.sync_copy(sh_disr.at[pl.ds(base, NSLICE)], deg_t)

    @pl.loop(0, NSLICE // 16)
    def _(i):
        sl = pl.ds(i * 16, 16)
        deg_t[sl] = _rsqrt16(deg_t[sl])

    pltpu.sync_copy(deg_t, sh_disr.at[pl.ds(base, NSLICE)])
    plsc.subcore_barrier()

    pltpu.sync_copy(sh_disr, disr_t)

    for seg in range(NSEG):
        pltpu.sync_copy(ei5.at[0, w, seg], rowi)
        pltpu.sync_copy(ei5.at[1, w, seg], coli)
        pltpu.sync_copy(dist5.at[w, seg], vals_t)

        @pl.loop(0, SEG)
        def _(i):
            for q in range(K // 16):
                sl = pl.ds(q * 16, 16)
                dd = vals_t[i, sl]
                dr = plsc.load_gather(disr_t, [rowi[i, sl]])
                dc = plsc.load_gather(disr_t, [coli[i, sl]])
                vals_t[i, sl] = jnp.exp(-dd * dd) * dr * dc

        pltpu.async_copy(x_hbm.at[coli.at[0]], ring.at[0], sem_g)

        @pl.loop(0, SEG)
        def _(i):
            p = lax.rem(i, 2)

            @pl.when(i >= 1)
            def _():
                pltpu.make_async_copy(ring.at[p], sh_out.at[rowi.at[0]],
                                      sem_s).wait()

            @pl.when(i < SEG - 1)
            def _():
                pltpu.async_copy(x_hbm.at[coli.at[i + 1]],
                                 ring.at[1 - p], sem_g)

            pltpu.make_async_copy(x_hbm.at[coli.at[i]], ring.at[p],
                                  sem_g).wait()

            for g in range(K // 16):
                vv = vals_t[i, pl.ds(g * 16, 16)]
                for j in range(16):
                    v = vv[j]
                    e = g * 16 + j
                    for k in range(D // 16):
                        sl = pl.ds(k * 16, 16)
                        ring[p, e, sl] = ring[p, e, sl] * v

            pltpu.async_copy(ring.at[p], sh_out.at[rowi.at[i]], sem_s,
                             add=True)

        pltpu.make_async_copy(ring.at[0], sh_out.at[rowi.at[0]], sem_s).wait()

    plsc.subcore_barrier()

    pltpu.sync_copy(sh_out.at[pl.ds(base, NSLICE)],
                    part_out.at[c, pl.ds(base, NSLICE)])


_sc_call = functools.partial(
    pl.kernel,
    out_type=jax.ShapeDtypeStruct((NC, NPAD, D), jnp.float32),
    mesh=plsc.VectorSubcoreMesh(core_axis_name="c", subcore_axis_name="s",
                                num_cores=NC, num_subcores=NS),
    compiler_params=pltpu.CompilerParams(needs_layout_passes=False),
    scratch_types=[
        pltpu.VMEM_SHARED((NPAD, D), jnp.float32),
        pltpu.VMEM_SHARED((NPAD,), jnp.float32),
        pltpu.VMEM((NPAD,), jnp.float32),
        pltpu.VMEM((SEG, K), jnp.int32),
        pltpu.VMEM((SEG, K), jnp.int32),
        pltpu.VMEM((SEG, K), jnp.float32),
        pltpu.VMEM((2, K, D), jnp.float32),
        pltpu.VMEM((NSLICE,), jnp.float32),
        pltpu.VMEM((K,), jnp.float32),
        pltpu.SemaphoreType.DMA,
        pltpu.SemaphoreType.DMA,
        pltpu.SemaphoreType.DMA,
    ],
)(_sc_body)


def _mm_body(p_ref, wt_ref, b_ref, o_ref):
    sblk = p_ref[0] + p_ref[1]
    o_ref[...] = (
        jnp.dot(sblk, wt_ref[...], preferred_element_type=jnp.float32)
        + b_ref[...]
    )


def _mm_call(parts, wt, b2):
    return pl.pallas_call(
        _mm_body,
        grid=(NPAD // 512,),
        in_specs=[
            pl.BlockSpec((NC, 512, D), lambda i: (0, i, 0)),
            pl.BlockSpec((D, D), lambda i: (0, 0)),
            pl.BlockSpec((1, D), lambda i: (0, 0)),
        ],
        out_specs=pl.BlockSpec((512, D), lambda i: (i, 0)),
        out_shape=jax.ShapeDtypeStruct((N, D), jnp.float32),
    )(parts, wt, b2)


@jax.jit
def kernel(x, edge_index, dist_vec, W, b):
    ei5 = edge_index.reshape(2, NW, NSEG, SEG, K)
    dist5 = dist_vec.reshape(NW, NSEG, SEG, K)
    parts = _sc_call(x, ei5, dist5)
    return _mm_call(parts, W.T, b.reshape(1, D))


# speedup vs baseline: 2.4895x; 1.0222x over previous
import functools

import jax
import jax.numpy as jnp
from jax import lax
from jax.experimental import pallas as pl
from jax.experimental.pallas import tpu as pltpu
from jax.experimental.pallas import tpu_sc as plsc

N = 10000
E = 320000
D = 128

NC = 2
NS = 16
NW = NC * NS
EP = E // NW
K = 80
SEG = 25
NSEG = EP // (SEG * K)
NPAD = 10240
NSLICE = NPAD // NS


def _rsqrt16(d):
    dm = jnp.maximum(d, 1.0)
    bits = lax.bitcast_convert_type(dm, jnp.int32)
    y = lax.bitcast_convert_type(jnp.int32(0x5F3759DF) - (bits >> 1),
                                 jnp.float32)
    for _ in range(3):
        y = y * (1.5 - 0.5 * dm * y * y)
    return jnp.where(d > 0.5, y, 0.0)


def _sc_body(x_hbm, ei5, dist5, part_out,
             sh_out, sh_disr,
             disr_t, rowi, coli, vals_t, ring,
             deg_t, ones_k, sem_g, sem_s, sem_d):
    _Z16 = jnp.zeros((16,), jnp.float32)
    _O16 = jnp.ones((16,), jnp.float32)
    c = lax.axis_index("c")
    s = lax.axis_index("s")
    w = 2 * s + c
    base = s * NSLICE

    @pl.loop(0, NSLICE // 16)
    def _(i):
        deg_t[pl.ds(i * 16, 16)] = _Z16

    @pl.loop(0, K)
    def _(r):
        for k in range(D // 16):
            ring[0, r, pl.ds(k * 16, 16)] = _Z16

    for q in range(K // 16):
        ones_k[pl.ds(q * 16, 16)] = _O16

    pltpu.sync_copy(deg_t, sh_disr.at[pl.ds(base, NSLICE)])
    for j in range(NSLICE // K):
        pltpu.sync_copy(ring.at[0], sh_out.at[pl.ds(base + j * K, K)])
    plsc.subcore_barrier()

    for seg in range(NSEG):
        pltpu.sync_copy(ei5.at[1, 2 * s, seg], rowi)
        pltpu.sync_copy(ei5.at[1, 2 * s + 1, seg], coli)

        @pl.loop(0, SEG)
        def _(i):
            pltpu.async_copy(ones_k, sh_disr.at[rowi.at[i]], sem_d, add=True)
            pltpu.async_copy(ones_k, sh_disr.at[coli.at[i]], sem_d, add=True)

        @pl.loop(0, 2 * SEG)
        def _(i):
            pltpu.make_async_copy(ones_k, sh_disr.at[coli.at[0]],
                                  sem_d).wait()

    plsc.subcore_barrier()

    pltpu.sync_copy(sh_disr.at[pl.ds(base, NSLICE)], deg_t)

    @pl.loop(0, NSLICE // 16)
    def _(i):
        sl = pl.ds(i * 16, 16)
        deg_t[sl] = _rsqrt16(deg_t[sl])

    pltpu.sync_copy(deg_t, sh_disr.at[pl.ds(base, NSLICE)])
    plsc.subcore_barrier()

    pltpu.sync_copy(sh_disr, disr_t)

    for seg in range(NSEG):
        pltpu.sync_copy(ei5.at[0, w, seg], rowi)
        pltpu.sync_copy(ei5.at[1, w, seg], coli)
        pltpu.sync_copy(dist5.at[w, seg], vals_t)

        pltpu.async_copy(x_hbm.at[coli.at[0]], ring.at[0], sem_g)

        @pl.loop(0, SEG)
        def _(i):
            p = lax.rem(i, 2)

            @pl.when(i >= 1)
            def _():
                pltpu.make_async_copy(ring.at[p], sh_out.at[rowi.at[0]],
                                      sem_s).wait()

            @pl.when(i < SEG - 1)
            def _():
                pltpu.async_copy(x_hbm.at[coli.at[i + 1]],
                                 ring.at[1 - p], sem_g)

            for q in range(K // 16):
                sl = pl.ds(q * 16, 16)
                dd = vals_t[i, sl]
                dr = plsc.load_gather(disr_t, [rowi[i, sl]])
                dc = plsc.load_gather(disr_t, [coli[i, sl]])
                vals_t[i, sl] = jnp.exp(-dd * dd) * dr * dc

            pltpu.make_async_copy(x_hbm.at[coli.at[i]], ring.at[p],
                                  sem_g).wait()

            for g in range(K // 16):
                vv = vals_t[i, pl.ds(g * 16, 16)]
                for j in range(16):
                    v = vv[j]
                    e = g * 16 + j
                    for k in range(D // 16):
                        sl = pl.ds(k * 16, 16)
                        ring[p, e, sl] = ring[p, e, sl] * v

            pltpu.async_copy(ring.at[p], sh_out.at[rowi.at[i]], sem_s,
                             add=True)

        pltpu.make_async_copy(ring.at[0], sh_out.at[rowi.at[0]], sem_s).wait()

    plsc.subcore_barrier()

    pltpu.sync_copy(sh_out.at[pl.ds(base, NSLICE)],
                    part_out.at[c, pl.ds(base, NSLICE)])


_sc_call = functools.partial(
    pl.kernel,
    out_type=jax.ShapeDtypeStruct((NC, NPAD, D), jnp.float32),
    mesh=plsc.VectorSubcoreMesh(core_axis_name="c", subcore_axis_name="s",
                                num_cores=NC, num_subcores=NS),
    compiler_params=pltpu.CompilerParams(needs_layout_passes=False),
    scratch_types=[
        pltpu.VMEM_SHARED((NPAD, D), jnp.float32),
        pltpu.VMEM_SHARED((NPAD,), jnp.float32),
        pltpu.VMEM((NPAD,), jnp.float32),
        pltpu.VMEM((SEG, K), jnp.int32),
        pltpu.VMEM((SEG, K), jnp.int32),
        pltpu.VMEM((SEG, K), jnp.float32),
        pltpu.VMEM((2, K, D), jnp.float32),
        pltpu.VMEM((NSLICE,), jnp.float32),
        pltpu.VMEM((K,), jnp.float32),
        pltpu.SemaphoreType.DMA,
        pltpu.SemaphoreType.DMA,
        pltpu.SemaphoreType.DMA,
    ],
)(_sc_body)


def _mm_body(p_ref, wt_ref, b_ref, o_ref):
    sblk = p_ref[0] + p_ref[1]
    o_ref[...] = (
        jnp.dot(sblk, wt_ref[...], preferred_element_type=jnp.float32)
        + b_ref[...]
    )


def _mm_call(parts, wt, b2):
    return pl.pallas_call(
        _mm_body,
        grid=(NPAD // 512,),
        in_specs=[
            pl.BlockSpec((NC, 512, D), lambda i: (0, i, 0)),
            pl.BlockSpec((D, D), lambda i: (0, 0)),
            pl.BlockSpec((1, D), lambda i: (0, 0)),
        ],
        out_specs=pl.BlockSpec((512, D), lambda i: (i, 0)),
        out_shape=jax.ShapeDtypeStruct((N, D), jnp.float32),
    )(parts, wt, b2)


@jax.jit
def kernel(x, edge_index, dist_vec, W, b):
    ei5 = edge_index.reshape(2, NW, NSEG, SEG, K)
    dist5 = dist_vec.reshape(NW, NSEG, SEG, K)
    parts = _sc_call(x, ei5, dist5)
    return _mm_call(parts, W.T, b.reshape(1, D))


# speedup vs baseline: 2.7227x; 1.0937x over previous
import functools

import jax
import jax.numpy as jnp
from jax import lax
from jax.experimental import pallas as pl
from jax.experimental.pallas import tpu as pltpu
from jax.experimental.pallas import tpu_sc as plsc

N = 10000
E = 320000
D = 128

NC = 2
NS = 16
NW = NC * NS
EP = E // NW
K = 80
SEG = 25
NSEG = EP // (SEG * K)
NPAD = 10240
NB = 8
DEPTH = 4
NSLICE = NPAD // NS


def _rsqrt16(d):
    dm = jnp.maximum(d, 1.0)
    bits = lax.bitcast_convert_type(dm, jnp.int32)
    y = lax.bitcast_convert_type(jnp.int32(0x5F3759DF) - (bits >> 1),
                                 jnp.float32)
    for _ in range(3):
        y = y * (1.5 - 0.5 * dm * y * y)
    return jnp.where(d > 0.5, y, 0.0)


def _sc_body(x_hbm, ei5, dist5, part_out,
             sh_out, sh_disr,
             disr_t, rowi, coli, vals_t, ring,
             deg_t, ones_k, sem_g, sem_s, sem_d):
    _Z16 = jnp.zeros((16,), jnp.float32)
    _O16 = jnp.ones((16,), jnp.float32)
    c = lax.axis_index("c")
    s = lax.axis_index("s")
    w = 2 * s + c
    base = s * NSLICE

    @pl.loop(0, NSLICE // 16)
    def _(i):
        deg_t[pl.ds(i * 16, 16)] = _Z16

    @pl.loop(0, 16)
    def _(r):
        for k in range(D // 16):
            ring[0, r, pl.ds(k * 16, 16)] = _Z16

    for q in range(K // 16):
        ones_k[pl.ds(q * 16, 16)] = _O16

    pltpu.sync_copy(deg_t, sh_disr.at[pl.ds(base, NSLICE)])
    @pl.loop(0, NSLICE // 16)
    def _(j):
        pltpu.sync_copy(ring.at[0], sh_out.at[pl.ds(base + j * 16, 16)])
    plsc.subcore_barrier()

    for seg in range(NSEG):
        pltpu.sync_copy(ei5.at[1, 2 * s, seg], rowi)
        pltpu.sync_copy(ei5.at[1, 2 * s + 1, seg], coli)

        @pl.loop(0, SEG)
        def _(i):
            pltpu.async_copy(ones_k, sh_disr.at[rowi.at[i]], sem_d, add=True)
            pltpu.async_copy(ones_k, sh_disr.at[coli.at[i]], sem_d, add=True)

        @pl.loop(0, 2 * SEG)
        def _(i):
            pltpu.make_async_copy(ones_k, sh_disr.at[coli.at[0]],
                                  sem_d).wait()

    plsc.subcore_barrier()

    pltpu.sync_copy(sh_disr.at[pl.ds(base, NSLICE)], deg_t)

    @pl.loop(0, NSLICE // 16)
    def _(i):
        sl = pl.ds(i * 16, 16)
        deg_t[sl] = _rsqrt16(deg_t[sl])

    pltpu.sync_copy(deg_t, sh_disr.at[pl.ds(base, NSLICE)])
    plsc.subcore_barrier()

    pltpu.sync_copy(sh_disr, disr_t)

    for seg in range(NSEG):
        pltpu.sync_copy(ei5.at[0, w, seg], rowi)
        pltpu.sync_copy(ei5.at[1, w, seg], coli)
        pltpu.sync_copy(dist5.at[w, seg], vals_t)

        def _cidx(i):
            return lax.div(i, 5), lax.rem(i, 5)

        for pre in range(DEPTH):
            ri, gi = _cidx(jnp.int32(pre))
            cc = coli[ri, pl.ds(gi * 16, 16)]
            pltpu.async_copy(x_hbm.at[cc], ring.at[pre], sem_g)

        @pl.loop(0, SEG * (K // 16))
        def _(i):
            m = lax.rem(i, NB)
            ri, gi = _cidx(i)
            sl = pl.ds(gi * 16, 16)
            rr = rowi[ri, sl]

            @pl.when(i >= NB - DEPTH)
            def _():
                pltpu.make_async_copy(ring.at[m], sh_out.at[rr],
                                      sem_s).wait()

            @pl.when(i < SEG * (K // 16) - DEPTH)
            def _():
                ri2, gi2 = _cidx(i + DEPTH)
                cc2 = coli[ri2, pl.ds(gi2 * 16, 16)]
                pltpu.async_copy(x_hbm.at[cc2],
                                 ring.at[lax.rem(i + DEPTH, NB)], sem_g)

            dd = vals_t[ri, sl]
            dr = plsc.load_gather(disr_t, [rr])
            dc = plsc.load_gather(disr_t, [coli[ri, sl]])
            vv = jnp.exp(-dd * dd) * dr * dc

            cc = coli[ri, sl]
            pltpu.make_async_copy(x_hbm.at[cc], ring.at[m], sem_g).wait()

            for j in range(16):
                v = vv[j]
                for k in range(D // 16):
                    slk = pl.ds(k * 16, 16)
                    ring[m, j, slk] = ring[m, j, slk] * v

            pltpu.async_copy(ring.at[m], sh_out.at[rr], sem_s, add=True)

        ri_l, gi_l = _cidx(jnp.int32(SEG * (K // 16) - 1))
        rr_l = rowi[ri_l, pl.ds(gi_l * 16, 16)]
        for _d in range(NB - DEPTH):
            pltpu.make_async_copy(ring.at[0], sh_out.at[rr_l], sem_s).wait()

    plsc.subcore_barrier()

    pltpu.sync_copy(sh_out.at[pl.ds(base, NSLICE)],
                    part_out.at[c, pl.ds(base, NSLICE)])


_sc_call = functools.partial(
    pl.kernel,
    out_type=jax.ShapeDtypeStruct((NC, NPAD, D), jnp.float32),
    mesh=plsc.VectorSubcoreMesh(core_axis_name="c", subcore_axis_name="s",
                                num_cores=NC, num_subcores=NS),
    compiler_params=pltpu.CompilerParams(needs_layout_passes=False),
    scratch_types=[
        pltpu.VMEM_SHARED((NPAD, D), jnp.float32),
        pltpu.VMEM_SHARED((NPAD,), jnp.float32),
        pltpu.VMEM((NPAD,), jnp.float32),
        pltpu.VMEM((SEG, K), jnp.int32),
        pltpu.VMEM((SEG, K), jnp.int32),
        pltpu.VMEM((SEG, K), jnp.float32),
        pltpu.VMEM((NB, 16, D), jnp.float32),
        pltpu.VMEM((NSLICE,), jnp.float32),
        pltpu.VMEM((K,), jnp.float32),
        pltpu.SemaphoreType.DMA,
        pltpu.SemaphoreType.DMA,
        pltpu.SemaphoreType.DMA,
    ],
)(_sc_body)


def _mm_body(p_ref, wt_ref, b_ref, o_ref):
    sblk = p_ref[0] + p_ref[1]
    o_ref[...] = (
        jnp.dot(sblk, wt_ref[...], preferred_element_type=jnp.float32)
        + b_ref[...]
    )


def _mm_call(parts, wt, b2):
    return pl.pallas_call(
        _mm_body,
        grid=(NPAD // 512,),
        in_specs=[
            pl.BlockSpec((NC, 512, D), lambda i: (0, i, 0)),
            pl.BlockSpec((D, D), lambda i: (0, 0)),
            pl.BlockSpec((1, D), lambda i: (0, 0)),
        ],
        out_specs=pl.BlockSpec((512, D), lambda i: (i, 0)),
        out_shape=jax.ShapeDtypeStruct((N, D), jnp.float32),
    )(parts, wt, b2)


@jax.jit
def kernel(x, edge_index, dist_vec, W, b):
    ei5 = edge_index.reshape(2, NW, NSEG, SEG, K)
    dist5 = dist_vec.reshape(NW, NSEG, SEG, K)
    parts = _sc_call(x, ei5, dist5)
    return _mm_call(parts, W.T, b.reshape(1, D))


# speedup vs baseline: 2.8938x; 1.0628x over previous
import functools

import jax
import jax.numpy as jnp
from jax import lax
from jax.experimental import pallas as pl
from jax.experimental.pallas import tpu as pltpu
from jax.experimental.pallas import tpu_sc as plsc

N = 10000
E = 320000
D = 128

NC = 2
NS = 16
NW = NC * NS
EP = E // NW
K = 80
SEG = 25
NSEG = EP // (SEG * K)
NPAD = 10240
NB = 8
DEPTH = 6
NSLICE = NPAD // NS


def _rsqrt16(d):
    dm = jnp.maximum(d, 1.0)
    bits = lax.bitcast_convert_type(dm, jnp.int32)
    y = lax.bitcast_convert_type(jnp.int32(0x5F3759DF) - (bits >> 1),
                                 jnp.float32)
    for _ in range(3):
        y = y * (1.5 - 0.5 * dm * y * y)
    return jnp.where(d > 0.5, y, 0.0)


def _sc_body(x_hbm, ei5, dist5, part_out,
             sh_out, sh_disr,
             disr_t, rowi, coli, vals_t, ring,
             deg_t, ones_k, sem_g, sem_s, sem_d):
    _Z16 = jnp.zeros((16,), jnp.float32)
    _O16 = jnp.ones((16,), jnp.float32)
    c = lax.axis_index("c")
    s = lax.axis_index("s")
    w = 2 * s + c
    base = s * NSLICE

    @pl.loop(0, NSLICE // 16)
    def _(i):
        deg_t[pl.ds(i * 16, 16)] = _Z16

    @pl.loop(0, 16)
    def _(r):
        for k in range(D // 16):
            ring[0, r, pl.ds(k * 16, 16)] = _Z16

    for q in range(K // 16):
        ones_k[pl.ds(q * 16, 16)] = _O16

    pltpu.sync_copy(deg_t, sh_disr.at[pl.ds(base, NSLICE)])
    @pl.loop(0, NSLICE // 16)
    def _(j):
        pltpu.sync_copy(ring.at[0], sh_out.at[pl.ds(base + j * 16, 16)])
    plsc.subcore_barrier()

    for seg in range(NSEG):
        pltpu.sync_copy(ei5.at[1, 2 * s, seg], rowi)
        pltpu.sync_copy(ei5.at[1, 2 * s + 1, seg], coli)

        @pl.loop(0, SEG)
        def _(i):
            pltpu.async_copy(ones_k, sh_disr.at[rowi.at[i]], sem_d, add=True)
            pltpu.async_copy(ones_k, sh_disr.at[coli.at[i]], sem_d, add=True)

        @pl.loop(0, 2 * SEG)
        def _(i):
            pltpu.make_async_copy(ones_k, sh_disr.at[coli.at[0]],
                                  sem_d).wait()

    plsc.subcore_barrier()

    pltpu.sync_copy(sh_disr.at[pl.ds(base, NSLICE)], deg_t)

    @pl.loop(0, NSLICE // 16)
    def _(i):
        sl = pl.ds(i * 16, 16)
        deg_t[sl] = _rsqrt16(deg_t[sl])

    pltpu.sync_copy(deg_t, sh_disr.at[pl.ds(base, NSLICE)])
    plsc.subcore_barrier()

    pltpu.sync_copy(sh_disr, disr_t)

    for seg in range(NSEG):
        pltpu.sync_copy(ei5.at[0, w, seg], rowi)
        pltpu.sync_copy(ei5.at[1, w, seg], coli)
        pltpu.sync_copy(dist5.at[w, seg], vals_t)

        def _cidx(i):
            return lax.div(i, 5), lax.rem(i, 5)

        for pre in range(DEPTH):
            ri, gi = _cidx(jnp.int32(pre))
            cc = coli[ri, pl.ds(gi * 16, 16)]
            pltpu.async_copy(x_hbm.at[cc], ring.at[pre], sem_g)

        @pl.loop(0, SEG * (K // 16))
        def _(i):
            m = lax.rem(i, NB)
            ri, gi = _cidx(i)
            sl = pl.ds(gi * 16, 16)
            rr = rowi[ri, sl]

            @pl.when(i >= NB - DEPTH)
            def _():
                pltpu.make_async_copy(ring.at[m], sh_out.at[rr],
                                      sem_s).wait()

            @pl.when(i < SEG * (K // 16) - DEPTH)
            def _():
                ri2, gi2 = _cidx(i + DEPTH)
                cc2 = coli[ri2, pl.ds(gi2 * 16, 16)]
                pltpu.async_copy(x_hbm.at[cc2],
                                 ring.at[lax.rem(i + DEPTH, NB)], sem_g)

            dd = vals_t[ri, sl]
            dr = plsc.load_gather(disr_t, [rr])
            dc = plsc.load_gather(disr_t, [coli[ri, sl]])
            vv = jnp.exp(-dd * dd) * dr * dc

            cc = coli[ri, sl]
            pltpu.make_async_copy(x_hbm.at[cc], ring.at[m], sem_g).wait()

            for j in range(16):
                v = vv[j]
                for k in range(D // 16):
                    slk = pl.ds(k * 16, 16)
                    ring[m, j, slk] = ring[m, j, slk] * v

            pltpu.async_copy(ring.at[m], sh_out.at[rr], sem_s, add=True)

        ri_l, gi_l = _cidx(jnp.int32(SEG * (K // 16) - 1))
        rr_l = rowi[ri_l, pl.ds(gi_l * 16, 16)]
        for _d in range(NB - DEPTH):
            pltpu.make_async_copy(ring.at[0], sh_out.at[rr_l], sem_s).wait()

    plsc.subcore_barrier()

    pltpu.sync_copy(sh_out.at[pl.ds(base, NSLICE)],
                    part_out.at[c, pl.ds(base, NSLICE)])


_sc_call = functools.partial(
    pl.kernel,
    out_type=jax.ShapeDtypeStruct((NC, NPAD, D), jnp.float32),
    mesh=plsc.VectorSubcoreMesh(core_axis_name="c", subcore_axis_name="s",
                                num_cores=NC, num_subcores=NS),
    compiler_params=pltpu.CompilerParams(needs_layout_passes=False),
    scratch_types=[
        pltpu.VMEM_SHARED((NPAD, D), jnp.float32),
        pltpu.VMEM_SHARED((NPAD,), jnp.float32),
        pltpu.VMEM((NPAD,), jnp.float32),
        pltpu.VMEM((SEG, K), jnp.int32),
        pltpu.VMEM((SEG, K), jnp.int32),
        pltpu.VMEM((SEG, K), jnp.float32),
        pltpu.VMEM((NB, 16, D), jnp.float32),
        pltpu.VMEM((NSLICE,), jnp.float32),
        pltpu.VMEM((K,), jnp.float32),
        pltpu.SemaphoreType.DMA,
        pltpu.SemaphoreType.DMA,
        pltpu.SemaphoreType.DMA,
    ],
)(_sc_body)


def _mm_body(p_ref, wt_ref, b_ref, o_ref):
    sblk = p_ref[0] + p_ref[1]
    o_ref[...] = (
        jnp.dot(sblk, wt_ref[...], preferred_element_type=jnp.float32)
        + b_ref[...]
    )


def _mm_call(parts, wt, b2):
    return pl.pallas_call(
        _mm_body,
        grid=(NPAD // 512,),
        in_specs=[
            pl.BlockSpec((NC, 512, D), lambda i: (0, i, 0)),
            pl.BlockSpec((D, D), lambda i: (0, 0)),
            pl.BlockSpec((1, D), lambda i: (0, 0)),
        ],
        out_specs=pl.BlockSpec((512, D), lambda i: (i, 0)),
        out_shape=jax.ShapeDtypeStruct((N, D), jnp.float32),
    )(parts, wt, b2)


@jax.jit
def kernel(x, edge_index, dist_vec, W, b):
    ei5 = edge_index.reshape(2, NW, NSEG, SEG, K)
    dist5 = dist_vec.reshape(NW, NSEG, SEG, K)
    parts = _sc_call(x, ei5, dist5)
    return _mm_call(parts, W.T, b.reshape(1, D))


# speedup vs baseline: 2.9029x; 1.0031x over previous
import functools

import jax
import jax.numpy as jnp
from jax import lax
from jax.experimental import pallas as pl
from jax.experimental.pallas import tpu as pltpu
from jax.experimental.pallas import tpu_sc as plsc

N = 10000
E = 320000
D = 128

NC = 2
NS = 16
NW = NC * NS
EP = E // NW
K = 80
SEG = 25
NSEG = EP // (SEG * K)
NPAD = 10240
NB = 10
DEPTH = 7
NSLICE = NPAD // NS


def _rsqrt16(d):
    dm = jnp.maximum(d, 1.0)
    bits = lax.bitcast_convert_type(dm, jnp.int32)
    y = lax.bitcast_convert_type(jnp.int32(0x5F3759DF) - (bits >> 1),
                                 jnp.float32)
    for _ in range(3):
        y = y * (1.5 - 0.5 * dm * y * y)
    return jnp.where(d > 0.5, y, 0.0)


def _sc_body(x_hbm, ei5, dist5, part_out,
             sh_out, sh_disr,
             disr_t, rowi, coli, vals_t, ring,
             deg_t, ones_k, sem_g, sem_s, sem_d):
    _Z16 = jnp.zeros((16,), jnp.float32)
    _O16 = jnp.ones((16,), jnp.float32)
    c = lax.axis_index("c")
    s = lax.axis_index("s")
    w = 2 * s + c
    base = s * NSLICE

    @pl.loop(0, NSLICE // 16)
    def _(i):
        deg_t[pl.ds(i * 16, 16)] = _Z16

    @pl.loop(0, 16)
    def _(r):
        for k in range(D // 16):
            ring[0, r, pl.ds(k * 16, 16)] = _Z16

    for q in range(K // 16):
        ones_k[pl.ds(q * 16, 16)] = _O16

    pltpu.sync_copy(deg_t, sh_disr.at[pl.ds(base, NSLICE)])
    @pl.loop(0, NSLICE // 16)
    def _(j):
        pltpu.sync_copy(ring.at[0], sh_out.at[pl.ds(base + j * 16, 16)])
    plsc.subcore_barrier()

    for seg in range(NSEG):
        pltpu.sync_copy(ei5.at[1, 2 * s, seg], rowi)
        pltpu.sync_copy(ei5.at[1, 2 * s + 1, seg], coli)

        @pl.loop(0, SEG)
        def _(i):
            pltpu.async_copy(ones_k, sh_disr.at[rowi.at[i]], sem_d, add=True)
            pltpu.async_copy(ones_k, sh_disr.at[coli.at[i]], sem_d, add=True)

        @pl.loop(0, 2 * SEG)
        def _(i):
            pltpu.make_async_copy(ones_k, sh_disr.at[coli.at[0]],
                                  sem_d).wait()

    plsc.subcore_barrier()

    pltpu.sync_copy(sh_disr.at[pl.ds(base, NSLICE)], deg_t)

    @pl.loop(0, NSLICE // 16)
    def _(i):
        sl = pl.ds(i * 16, 16)
        deg_t[sl] = _rsqrt16(deg_t[sl])

    pltpu.sync_copy(deg_t, sh_disr.at[pl.ds(base, NSLICE)])
    plsc.subcore_barrier()

    pltpu.sync_copy(sh_disr, disr_t)

    for seg in range(NSEG):
        pltpu.sync_copy(ei5.at[0, w, seg], rowi)
        pltpu.sync_copy(ei5.at[1, w, seg], coli)
        pltpu.sync_copy(dist5.at[w, seg], vals_t)

        def _cidx(i):
            return lax.div(i, 5), lax.rem(i, 5)

        for pre in range(DEPTH):
            ri, gi = _cidx(jnp.int32(pre))
            cc = coli[ri, pl.ds(gi * 16, 16)]
            pltpu.async_copy(x_hbm.at[cc], ring.at[pre], sem_g)

        @pl.loop(0, SEG * (K // 16))
        def _(i):
            m = lax.rem(i, NB)
            ri, gi = _cidx(i)
            sl = pl.ds(gi * 16, 16)
            rr = rowi[ri, sl]

            @pl.when(i >= NB - DEPTH)
            def _():
                pltpu.make_async_copy(ring.at[m], sh_out.at[rr],
                                      sem_s).wait()

            @pl.when(i < SEG * (K // 16) - DEPTH)
            def _():
                ri2, gi2 = _cidx(i + DEPTH)
                cc2 = coli[ri2, pl.ds(gi2 * 16, 16)]
                pltpu.async_copy(x_hbm.at[cc2],
                                 ring.at[lax.rem(i + DEPTH, NB)], sem_g)

            dd = vals_t[ri, sl]
            dr = plsc.load_gather(disr_t, [rr])
            dc = plsc.load_gather(disr_t, [coli[ri, sl]])
            vv = jnp.exp(-dd * dd) * dr * dc

            cc = coli[ri, sl]
            pltpu.make_async_copy(x_hbm.at[cc], ring.at[m], sem_g).wait()

            for j in range(16):
                v = vv[j]
                for k in range(D // 16):
                    slk = pl.ds(k * 16, 16)
                    ring[m, j, slk] = ring[m, j, slk] * v

            pltpu.async_copy(ring.at[m], sh_out.at[rr], sem_s, add=True)

        ri_l, gi_l = _cidx(jnp.int32(SEG * (K // 16) - 1))
        rr_l = rowi[ri_l, pl.ds(gi_l * 16, 16)]
        for _d in range(NB - DEPTH):
            pltpu.make_async_copy(ring.at[0], sh_out.at[rr_l], sem_s).wait()

    plsc.subcore_barrier()

    pltpu.sync_copy(sh_out.at[pl.ds(base, NSLICE)],
                    part_out.at[c, pl.ds(base, NSLICE)])


_sc_call = functools.partial(
    pl.kernel,
    out_type=jax.ShapeDtypeStruct((NC, NPAD, D), jnp.float32),
    mesh=plsc.VectorSubcoreMesh(core_axis_name="c", subcore_axis_name="s",
                                num_cores=NC, num_subcores=NS),
    compiler_params=pltpu.CompilerParams(needs_layout_passes=False),
    scratch_types=[
        pltpu.VMEM_SHARED((NPAD, D), jnp.float32),
        pltpu.VMEM_SHARED((NPAD,), jnp.float32),
        pltpu.VMEM((NPAD,), jnp.float32),
        pltpu.VMEM((SEG, K), jnp.int32),
        pltpu.VMEM((SEG, K), jnp.int32),
        pltpu.VMEM((SEG, K), jnp.float32),
        pltpu.VMEM((NB, 16, D), jnp.float32),
        pltpu.VMEM((NSLICE,), jnp.float32),
        pltpu.VMEM((K,), jnp.float32),
        pltpu.SemaphoreType.DMA,
        pltpu.SemaphoreType.DMA,
        pltpu.SemaphoreType.DMA,
    ],
)(_sc_body)


def _mm_body(p_ref, wt_ref, b_ref, o_ref):
    sblk = p_ref[0] + p_ref[1]
    o_ref[...] = (
        jnp.dot(sblk, wt_ref[...], preferred_element_type=jnp.float32)
        + b_ref[...]
    )


def _mm_call(parts, wt, b2):
    return pl.pallas_call(
        _mm_body,
        grid=(NPAD // 512,),
        in_specs=[
            pl.BlockSpec((NC, 512, D), lambda i: (0, i, 0)),
            pl.BlockSpec((D, D), lambda i: (0, 0)),
            pl.BlockSpec((1, D), lambda i: (0, 0)),
        ],
        out_specs=pl.BlockSpec((512, D), lambda i: (i, 0)),
        out_shape=jax.ShapeDtypeStruct((N, D), jnp.float32),
    )(parts, wt, b2)


@jax.jit
def kernel(x, edge_index, dist_vec, W, b):
    ei5 = edge_index.reshape(2, NW, NSEG, SEG, K)
    dist5 = dist_vec.reshape(NW, NSEG, SEG, K)
    parts = _sc_call(x, ei5, dist5)
    return _mm_call(parts, W.T, b.reshape(1, D))
